# Initial kernel scaffold; baseline (speedup 1.0000x reference)
import jax
import jax.numpy as jnp
from jax.experimental import pallas as pl


def kernel(outputs, labels, cost_matrix):
    raise NotImplementedError("write your pallas kernel here")


# speedup vs baseline: 1.9804x; 1.9804x over previous
import functools

import jax
import jax.numpy as jnp
from jax import lax
from jax.experimental import pallas as pl
from jax.experimental.pallas import tpu as pltpu
from jax.experimental.pallas import tpu_sc as plsc

_BB = 512
_CP = 1008


def _colmax_body(x_ref, o_ref):
    i = pl.program_id(0)
    x = x_ref[...]
    rowmax = jnp.max(x, axis=1, keepdims=True)
    cm = jnp.max(x - rowmax, axis=0, keepdims=True)

    @pl.when(i == 0)
    def _():
        o_ref[...] = cm

    @pl.when(i != 0)
    def _():
        o_ref[...] = jnp.maximum(o_ref[...], cm)


def _loss_body(cm_ref, cnt_ref, acc_ref, o_ref):
    c = cm_ref.shape[0]
    cnt = jnp.sum(cnt_ref[...], axis=0, keepdims=True)[:, :c]
    w = lax.dot_general(
        cnt,
        cm_ref[...],
        (((1,), (0,)), ((), ())),
        precision=lax.Precision.HIGHEST,
        preferred_element_type=jnp.float32,
    )
    mask = (acc_ref[...] == 0.0).astype(jnp.float32)
    o_ref[...] = -jnp.sum(w * mask, keepdims=True)


@functools.lru_cache(maxsize=None)
def _make_hist(n, nw, nc, nb):
    mesh = plsc.VectorSubcoreMesh(core_axis_name="c", subcore_axis_name="s")

    @functools.partial(
        pl.kernel,
        mesh=mesh,
        out_type=jax.ShapeDtypeStruct((nw, _CP), jnp.float32),
        scratch_types=[
            pltpu.VMEM((nb,), jnp.int32),
            pltpu.VMEM((_CP,), jnp.float32),
        ],
        compiler_params=pltpu.CompilerParams(needs_layout_passes=False),
    )
    def hist_k(labels_hbm, out_hbm, idx_v, hist_v):
        wid = lax.axis_index("s") * nc + lax.axis_index("c")
        pltpu.sync_copy(labels_hbm.at[pl.ds(wid * nb, nb)], idx_v)
        zeros = jnp.zeros((16,), jnp.float32)
        for j in range(_CP // 16):
            hist_v[pl.ds(j * 16, 16)] = zeros
        ones = jnp.ones((16,), jnp.float32)
        for i in range(nb // 16):
            plsc.addupdate_scatter(hist_v, [idx_v[pl.ds(i * 16, 16)]], ones)
        pltpu.sync_copy(hist_v, out_hbm.at[wid])

    return hist_k


def kernel(outputs, labels, cost_matrix):
    b, c = outputs.shape

    acc = pl.pallas_call(
        _colmax_body,
        grid=(b // _BB,),
        in_specs=[pl.BlockSpec((_BB, c), lambda i: (i, 0))],
        out_specs=pl.BlockSpec((1, c), lambda i: (0, 0)),
        out_shape=jax.ShapeDtypeStruct((1, c), jnp.float32),
    )(outputs)

    info = plsc.get_sparse_core_info()
    nw = info.num_cores * info.num_subcores
    cnt32 = _make_hist(b, nw, info.num_cores, b // nw)(labels)

    loss = pl.pallas_call(
        _loss_body,
        in_specs=[
            pl.BlockSpec((c, c), lambda: (0, 0)),
            pl.BlockSpec(cnt32.shape, lambda: (0, 0)),
            pl.BlockSpec((1, c), lambda: (0, 0)),
        ],
        out_specs=pl.BlockSpec((1, 1), lambda: (0, 0)),
        out_shape=jax.ShapeDtypeStruct((1, 1), jnp.float32),
    )(cost_matrix, cnt32, acc)

    return loss[0, 0]


# speedup vs baseline: 2.1881x; 1.1049x over previous
import functools

import jax
import jax.numpy as jnp
from jax import lax
from jax.experimental import pallas as pl
from jax.experimental.pallas import tpu as pltpu
from jax.experimental.pallas import tpu_sc as plsc

_BB = 2048
_CP = 1008


def _colmax_body(x_ref, o_ref):
    i = pl.program_id(0)
    x = x_ref[...]
    rowmax = jnp.max(x, axis=1, keepdims=True)
    cm = jnp.max(x - rowmax, axis=0, keepdims=True)

    @pl.when(i == 0)
    def _():
        o_ref[...] = cm

    @pl.when(i != 0)
    def _():
        o_ref[...] = jnp.maximum(o_ref[...], cm)


def _loss_body(cm_ref, cnt_ref, acc_ref, o_ref):
    c = cm_ref.shape[0]
    cnt = jnp.sum(cnt_ref[...], axis=0, keepdims=True)[:, :c]
    w = lax.dot_general(
        cnt,
        cm_ref[...],
        (((1,), (0,)), ((), ())),
        precision=lax.Precision.HIGHEST,
        preferred_element_type=jnp.float32,
    )
    mask = (acc_ref[...] == 0.0).astype(jnp.float32)
    o_ref[...] = -jnp.sum(w * mask, keepdims=True)


@functools.lru_cache(maxsize=None)
def _make_hist(n, nw, nc, nb):
    mesh = plsc.VectorSubcoreMesh(core_axis_name="c", subcore_axis_name="s")

    @functools.partial(
        pl.kernel,
        mesh=mesh,
        out_type=jax.ShapeDtypeStruct((nw, _CP), jnp.float32),
        scratch_types=[
            pltpu.VMEM((nb,), jnp.int32),
            pltpu.VMEM((_CP,), jnp.float32),
        ],
        compiler_params=pltpu.CompilerParams(needs_layout_passes=False),
    )
    def hist_k(labels_hbm, out_hbm, idx_v, hist_v):
        wid = lax.axis_index("s") * nc + lax.axis_index("c")
        pltpu.sync_copy(labels_hbm.at[pl.ds(wid * nb, nb)], idx_v)
        zeros = jnp.zeros((16,), jnp.float32)
        for j in range(_CP // 16):
            hist_v[pl.ds(j * 16, 16)] = zeros
        ones = jnp.ones((16,), jnp.float32)
        for i in range(nb // 16):
            plsc.addupdate_scatter(hist_v, [idx_v[pl.ds(i * 16, 16)]], ones)
        pltpu.sync_copy(hist_v, out_hbm.at[wid])

    return hist_k


def kernel(outputs, labels, cost_matrix):
    b, c = outputs.shape

    acc = pl.pallas_call(
        _colmax_body,
        grid=(b // _BB,),
        in_specs=[pl.BlockSpec((_BB, c), lambda i: (i, 0))],
        out_specs=pl.BlockSpec((1, c), lambda i: (0, 0)),
        out_shape=jax.ShapeDtypeStruct((1, c), jnp.float32),
    )(outputs)

    info = plsc.get_sparse_core_info()
    nw = info.num_cores * info.num_subcores
    cnt32 = _make_hist(b, nw, info.num_cores, b // nw)(labels)

    loss = pl.pallas_call(
        _loss_body,
        in_specs=[
            pl.BlockSpec((c, c), lambda: (0, 0)),
            pl.BlockSpec(cnt32.shape, lambda: (0, 0)),
            pl.BlockSpec((1, c), lambda: (0, 0)),
        ],
        out_specs=pl.BlockSpec((1, 1), lambda: (0, 0)),
        out_shape=jax.ShapeDtypeStruct((1, 1), jnp.float32),
    )(cost_matrix, cnt32, acc)

    return loss[0, 0]
